# baseline (speedup 1.0000x reference)
import jax
import jax.numpy as jnp
from jax.experimental import pallas as pl
from jax.experimental.pallas import tpu as pltpu

_ROWS_PER_BLOCK = 64


def _copy_body(x1_ref, x2_ref, o1_ref, o2_ref, sem1, sem2):
    i = pl.program_id(0)
    rows = pl.ds(i * _ROWS_PER_BLOCK, _ROWS_PER_BLOCK)
    c1 = pltpu.make_async_copy(x1_ref, o1_ref.at[rows], sem1)
    c2 = pltpu.make_async_copy(x2_ref, o2_ref.at[rows], sem2)
    c1.start()
    c2.start()
    c1.wait()
    c2.wait()


def kernel(x1, x2):
    N, C, H, W = x1.shape
    rows = N * C
    a = x1.reshape(rows, H, W)
    b = x2.reshape(rows, H, W)
    grid = (rows // _ROWS_PER_BLOCK,)
    in_spec = pl.BlockSpec((_ROWS_PER_BLOCK, H, W), lambda i: (i, 0, 0))
    out_spec = pl.BlockSpec(memory_space=pltpu.MemorySpace.HBM)
    out1, out2 = pl.pallas_call(
        _copy_body,
        grid=grid,
        out_shape=(
            jax.ShapeDtypeStruct((rows, H, W), x1.dtype),
            jax.ShapeDtypeStruct((rows, H, W), x2.dtype),
        ),
        in_specs=[in_spec, in_spec],
        out_specs=(out_spec, out_spec),
        scratch_shapes=[pltpu.SemaphoreType.DMA, pltpu.SemaphoreType.DMA],
    )(a, b)
    return (out1.reshape(N, C, H, W), out2.reshape(N, C, H, W))
